# baseline (speedup 1.0000x reference)
import jax
import jax.numpy as jnp
from jax import lax
from jax.experimental import pallas as pl
from jax.experimental.pallas import tpu as pltpu
from jax.experimental.pallas import tpu_sc as plsc

N = 10000
E = 160000
D = 512
H = 256
HH = H // 2

NC = 2
NS = 16
LANES = 16

R = 10240
ROWS_PER_TILE = R // NS
E_PAD = 163840
EPT = E_PAD // NS
CHUNK = 128
NCHUNK = EPT // CHUNK
SEG = 40
NSEG = NCHUNK // SEG
EPT_DEG = E_PAD // (NC * NS)

_MESH = plsc.VectorSubcoreMesh(
    core_axis_name="c", subcore_axis_name="s", num_cores=NC, num_subcores=NS
)

_SC_PARAMS = pltpu.CompilerParams(needs_layout_passes=False)


def _sc_degrees_body(srcd_hbm, dstd_hbm, dego_hbm, degi_hbm,
                     src_v, dst_v, dego_v, degi_v):
    cid = lax.axis_index("c")
    sid = lax.axis_index("s")
    wid = sid * NC + cid

    pltpu.sync_copy(srcd_hbm.at[wid], src_v)
    pltpu.sync_copy(dstd_hbm.at[wid], dst_v)

    zeros16 = jnp.zeros((LANES,), jnp.float32)

    def zero_body(i, _):
        dego_v[pl.ds(i * LANES, LANES)] = zeros16
        degi_v[pl.ds(i * LANES, LANES)] = zeros16
        return _

    lax.fori_loop(0, R // LANES, zero_body, None)

    ones16 = jnp.ones((LANES,), jnp.float32)

    def hist_body(i, _):
        s_idx = src_v[pl.ds(i * LANES, LANES)]
        plsc.addupdate_scatter(dego_v, [s_idx], ones16)
        d_idx = dst_v[pl.ds(i * LANES, LANES)]
        plsc.addupdate_scatter(degi_v, [d_idx], ones16)
        return _

    lax.fori_loop(0, EPT_DEG // LANES, hist_body, None)

    pltpu.sync_copy(dego_v, dego_hbm.at[wid])
    pltpu.sync_copy(degi_v, degi_hbm.at[wid])


_sc_degrees = pl.kernel(
    _sc_degrees_body,
    out_type=(
        jax.ShapeDtypeStruct((NC * NS, R), jnp.float32),
        jax.ShapeDtypeStruct((NC * NS, R), jnp.float32),
    ),
    mesh=_MESH,
    scratch_types=[
        pltpu.VMEM((EPT_DEG,), jnp.int32),
        pltpu.VMEM((EPT_DEG,), jnp.int32),
        pltpu.VMEM((R,), jnp.float32),
        pltpu.VMEM((R,), jnp.float32),
    ],
    compiler_params=_SC_PARAMS,
)


def _sc_propagate_body(table_hbm, srcp0_hbm, srcp1_hbm, dstp_hbm, agg_hbm,
                       src_v, dst_v, rows_v, rows_v1, acc,
                       sem, sem1, sem2, sem3):
    cid = lax.axis_index("c")
    sid = lax.axis_index("s")

    zeros16 = jnp.zeros((LANES,), jnp.float32)

    def zrow(i, _):
        r = i // (HH // LANES)
        k = i % (HH // LANES)
        rows_v[r, pl.ds(k * LANES, LANES)] = zeros16
        return _

    lax.fori_loop(0, CHUNK * (HH // LANES), zrow, None)

    base = sid * ROWS_PER_TILE
    for z in range(ROWS_PER_TILE // CHUNK):
        pltpu.sync_copy(rows_v, acc.at[pl.ds(base + z * CHUNK, CHUNK)])

    plsc.subcore_barrier()

    def wait_gather(buf, s):
        pltpu.make_async_copy(table_hbm.at[src_v.at[0]], buf, s).wait()

    def wait_scatter(buf, s):
        pltpu.make_async_copy(buf, acc.at[dst_v.at[0]], s).wait()

    def seg_body(sg, _):
        @pl.when(cid == 0)
        def _():
            pltpu.sync_copy(srcp0_hbm.at[sid, sg], src_v)

        @pl.when(cid == 1)
        def _():
            pltpu.sync_copy(srcp1_hbm.at[sid, sg], src_v)

        pltpu.sync_copy(dstp_hbm.at[sid, sg], dst_v)

        pltpu.async_copy(table_hbm.at[src_v.at[0]], rows_v, sem)
        pltpu.async_copy(table_hbm.at[src_v.at[1]], rows_v1, sem1)

        def edge_body(g, _):
            j0 = 2 * g
            wait_gather(rows_v, sem)
            pltpu.async_copy(rows_v, acc.at[dst_v.at[j0]], sem2, add=True)
            wait_gather(rows_v1, sem1)
            pltpu.async_copy(rows_v1, acc.at[dst_v.at[j0 + 1]], sem3, add=True)

            @pl.when(j0 + 2 < SEG)
            def _():
                wait_scatter(rows_v, sem2)
                pltpu.async_copy(table_hbm.at[src_v.at[j0 + 2]], rows_v, sem)

            @pl.when(j0 + 3 < SEG)
            def _():
                wait_scatter(rows_v1, sem3)
                pltpu.async_copy(table_hbm.at[src_v.at[j0 + 3]], rows_v1, sem1)

            return _

        lax.fori_loop(0, SEG // 2, edge_body, None)
        wait_scatter(rows_v, sem2)
        wait_scatter(rows_v1, sem3)
        return _

    lax.fori_loop(0, NSEG, seg_body, None)

    plsc.subcore_barrier()

    out_base = cid * R + base
    pltpu.sync_copy(acc.at[pl.ds(base, ROWS_PER_TILE)],
                    agg_hbm.at[pl.ds(out_base, ROWS_PER_TILE)])


_sc_propagate = pl.kernel(
    _sc_propagate_body,
    out_type=jax.ShapeDtypeStruct((NC * R, HH), jnp.float32),
    mesh=_MESH,
    scratch_types=[
        pltpu.VMEM((SEG, CHUNK), jnp.int32),
        pltpu.VMEM((SEG, CHUNK), jnp.int32),
        pltpu.VMEM((CHUNK, HH), jnp.float32),
        pltpu.VMEM((CHUNK, HH), jnp.float32),
        pltpu.VMEM_SHARED((R, HH), jnp.float32),
        pltpu.SemaphoreType.DMA,
        pltpu.SemaphoreType.DMA,
        pltpu.SemaphoreType.DMA,
        pltpu.SemaphoreType.DMA,
    ],
    compiler_params=_SC_PARAMS,
)


_BM = 1024
_GRID = R // _BM


def _dot(a, b):
    return jnp.dot(a, b, preferred_element_type=jnp.float32,
                   precision=lax.Precision.DEFAULT)


def _tc_finalize_deg_body(dego_ref, degi_ref, dinv_out_ref, dinv_in_ref):
    do = jnp.maximum(jnp.sum(dego_ref[...], axis=0), 1.0)
    di = jnp.maximum(jnp.sum(degi_ref[...], axis=0), 1.0)
    dinv_out_ref[...] = lax.rsqrt(do)[:, None]
    dinv_in_ref[...] = lax.rsqrt(di)[:, None]


def _tc_finalize_deg(dego_p, degi_p):
    return pl.pallas_call(
        _tc_finalize_deg_body,
        out_shape=(
            jax.ShapeDtypeStruct((R, 1), jnp.float32),
            jax.ShapeDtypeStruct((R, 1), jnp.float32),
        ),
    )(dego_p, degi_p)


def _tc_y1_body(x_ref, w_ref, dinv_ref, out_ref):
    y = _dot(x_ref[...], w_ref[...]) * dinv_ref[...]
    out_ref[0, :, :] = y[:, :HH]
    out_ref[1, :, :] = y[:, HH:]


def _tc_y1(feats_p, W_gc1, dinv_out):
    return pl.pallas_call(
        _tc_y1_body,
        grid=(_GRID,),
        in_specs=[
            pl.BlockSpec((_BM, D), lambda r: (r, 0)),
            pl.BlockSpec((D, H), lambda r: (0, 0)),
            pl.BlockSpec((_BM, 1), lambda r: (r, 0)),
        ],
        out_specs=pl.BlockSpec((NC, _BM, HH), lambda r: (0, r, 0)),
        out_shape=jax.ShapeDtypeStruct((NC, R, HH), jnp.float32),
    )(feats_p, W_gc1, dinv_out)


def _tc_mid_body(agg_ref, din_ref, dout_ref, b_ref, out_ref):
    x = jax.nn.relu(agg_ref[...] * din_ref[...][None] + b_ref[...])
    out_ref[...] = x * dout_ref[...][None]


def _tc_mid(agg1, dinv_in, dinv_out, b_gc1_2):
    return pl.pallas_call(
        _tc_mid_body,
        grid=(_GRID,),
        in_specs=[
            pl.BlockSpec((NC, _BM, HH), lambda r: (0, r, 0)),
            pl.BlockSpec((_BM, 1), lambda r: (r, 0)),
            pl.BlockSpec((_BM, 1), lambda r: (r, 0)),
            pl.BlockSpec((NC, 1, HH), lambda r: (0, 0, 0)),
        ],
        out_specs=pl.BlockSpec((NC, _BM, HH), lambda r: (0, r, 0)),
        out_shape=jax.ShapeDtypeStruct((NC, R, HH), jnp.float32),
    )(agg1, dinv_in, dinv_out, b_gc1_2)


def _tc_final_body(agg_ref, din_ref, feat_ref, wg2_ref, bg2_ref,
                   wm1_ref, bm1_ref, wm2_ref, bm2_ref, out_ref):
    a = jnp.concatenate([agg_ref[0], agg_ref[1]], axis=1) * din_ref[...]
    gcn = _dot(a, wg2_ref[...]) + bg2_ref[...] + feat_ref[...]
    m = jax.nn.relu(_dot(gcn, wm1_ref[...]) + bm1_ref[...])
    out_ref[...] = _dot(m, wm2_ref[...]) + bm2_ref[...] + gcn


def _tc_final(agg2, dinv_in, feats_p, W_gc2, b_gc2, W_m1, b_m1, W_m2, b_m2):
    return pl.pallas_call(
        _tc_final_body,
        grid=(_GRID,),
        in_specs=[
            pl.BlockSpec((NC, _BM, HH), lambda r: (0, r, 0)),
            pl.BlockSpec((_BM, 1), lambda r: (r, 0)),
            pl.BlockSpec((_BM, D), lambda r: (r, 0)),
            pl.BlockSpec((H, D), lambda r: (0, 0)),
            pl.BlockSpec((1, D), lambda r: (0, 0)),
            pl.BlockSpec((D, H), lambda r: (0, 0)),
            pl.BlockSpec((1, H), lambda r: (0, 0)),
            pl.BlockSpec((H, D), lambda r: (0, 0)),
            pl.BlockSpec((1, D), lambda r: (0, 0)),
        ],
        out_specs=pl.BlockSpec((_BM, D), lambda r: (r, 0)),
        out_shape=jax.ShapeDtypeStruct((R, D), jnp.float32),
    )(agg2, dinv_in, feats_p, W_gc2, b_gc2, W_m1, b_m1, W_m2, b_m2)


def kernel(features, edge_index, W_gc1, b_gc1, W_gc2, b_gc2,
           W_m1, b_m1, W_m2, b_m2):
    src = edge_index[0]
    dst = edge_index[1]
    pad = jnp.full((E_PAD - E,), N, dtype=jnp.int32)
    src_p = jnp.concatenate([src, pad])
    dst_p = jnp.concatenate([dst, pad])

    srcd = src_p.reshape(NC * NS, EPT_DEG)
    dstd = dst_p.reshape(NC * NS, EPT_DEG)
    srcp0 = src_p.reshape(NS, NSEG, SEG, CHUNK)
    srcp1 = srcp0 + R
    dstp = dst_p.reshape(NS, NSEG, SEG, CHUNK)

    feats_p = jnp.pad(features, ((0, R - N), (0, 0)))

    dego_p, degi_p = _sc_degrees(srcd, dstd)
    dinv_out, dinv_in = _tc_finalize_deg(dego_p, degi_p)

    y1 = _tc_y1(feats_p, W_gc1, dinv_out)
    agg1 = _sc_propagate(y1.reshape(NC * R, HH), srcp0, srcp1, dstp)
    h2 = _tc_mid(agg1.reshape(NC, R, HH), dinv_in, dinv_out,
                 b_gc1.reshape(NC, 1, HH))
    agg2 = _sc_propagate(h2.reshape(NC * R, HH), srcp0, srcp1, dstp)
    out_p = _tc_final(agg2.reshape(NC, R, HH), dinv_in, feats_p,
                      W_gc2, b_gc2.reshape(1, D), W_m1, b_m1.reshape(1, H),
                      W_m2, b_m2.reshape(1, D))
    return out_p[:N]


# speedup vs baseline: 1.0703x; 1.0703x over previous
import jax
import jax.numpy as jnp
from jax import lax
from jax.experimental import pallas as pl
from jax.experimental.pallas import tpu as pltpu
from jax.experimental.pallas import tpu_sc as plsc

N = 10000
E = 160000
D = 512
H = 256
HH = H // 2

NC = 2
NS = 16
LANES = 16

R = 10240
ROWS_PER_TILE = R // NS
E_PAD = 163840
EPT = E_PAD // NS
CHUNK = 128
NCHUNK = EPT // CHUNK
SEG = 40
NSEG = NCHUNK // SEG
EPT_DEG = E_PAD // (NC * NS)

_MESH = plsc.VectorSubcoreMesh(
    core_axis_name="c", subcore_axis_name="s", num_cores=NC, num_subcores=NS
)

_SC_PARAMS = pltpu.CompilerParams(needs_layout_passes=False)


def _sc_degrees_body(srcd_hbm, dstd_hbm, dego_hbm, degi_hbm,
                     src_v, dst_v, dego_v, degi_v):
    cid = lax.axis_index("c")
    sid = lax.axis_index("s")
    wid = sid * NC + cid

    pltpu.sync_copy(srcd_hbm.at[wid], src_v)
    pltpu.sync_copy(dstd_hbm.at[wid], dst_v)

    zeros16 = jnp.zeros((LANES,), jnp.float32)

    def zero_body(i, _):
        dego_v[pl.ds(i * LANES, LANES)] = zeros16
        degi_v[pl.ds(i * LANES, LANES)] = zeros16
        return _

    lax.fori_loop(0, R // LANES, zero_body, None)

    ones16 = jnp.ones((LANES,), jnp.float32)

    def hist_body(i, _):
        s_idx = src_v[pl.ds(i * LANES, LANES)]
        plsc.addupdate_scatter(dego_v, [s_idx], ones16)
        d_idx = dst_v[pl.ds(i * LANES, LANES)]
        plsc.addupdate_scatter(degi_v, [d_idx], ones16)
        return _

    lax.fori_loop(0, EPT_DEG // LANES, hist_body, None)

    pltpu.sync_copy(dego_v, dego_hbm.at[wid])
    pltpu.sync_copy(degi_v, degi_hbm.at[wid])


_sc_degrees = pl.kernel(
    _sc_degrees_body,
    out_type=(
        jax.ShapeDtypeStruct((NC * NS, R), jnp.float32),
        jax.ShapeDtypeStruct((NC * NS, R), jnp.float32),
    ),
    mesh=_MESH,
    scratch_types=[
        pltpu.VMEM((EPT_DEG,), jnp.int32),
        pltpu.VMEM((EPT_DEG,), jnp.int32),
        pltpu.VMEM((R,), jnp.float32),
        pltpu.VMEM((R,), jnp.float32),
    ],
    compiler_params=_SC_PARAMS,
)


def _sc_propagate_body(table_hbm, srcp0_hbm, srcp1_hbm, dstp_hbm, agg_hbm,
                       src_v, dst_v, rows_v, rows_v1, acc, sem, sem1):
    cid = lax.axis_index("c")
    sid = lax.axis_index("s")

    zeros16 = jnp.zeros((LANES,), jnp.float32)

    def zrow(i, _):
        r = i // (HH // LANES)
        k = i % (HH // LANES)
        rows_v[r, pl.ds(k * LANES, LANES)] = zeros16
        return _

    lax.fori_loop(0, CHUNK * (HH // LANES), zrow, None)

    base = sid * ROWS_PER_TILE
    for z in range(ROWS_PER_TILE // CHUNK):
        pltpu.sync_copy(rows_v, acc.at[pl.ds(base + z * CHUNK, CHUNK)])

    plsc.subcore_barrier()

    def wait_gather(buf, s):
        pltpu.make_async_copy(table_hbm.at[src_v.at[0]], buf, s).wait()

    def seg_body(sg, _):
        @pl.when(cid == 0)
        def _():
            pltpu.sync_copy(srcp0_hbm.at[sid, sg], src_v)

        @pl.when(cid == 1)
        def _():
            pltpu.sync_copy(srcp1_hbm.at[sid, sg], src_v)

        pltpu.sync_copy(dstp_hbm.at[sid, sg], dst_v)

        pltpu.async_copy(table_hbm.at[src_v.at[0]], rows_v, sem)

        def edge_body(g, _):
            j0 = 2 * g
            pltpu.async_copy(table_hbm.at[src_v.at[j0 + 1]], rows_v1, sem1)
            wait_gather(rows_v, sem)
            pltpu.sync_copy(rows_v, acc.at[dst_v.at[j0]], add=True)

            @pl.when(j0 + 2 < SEG)
            def _():
                pltpu.async_copy(table_hbm.at[src_v.at[j0 + 2]], rows_v, sem)

            wait_gather(rows_v1, sem1)
            pltpu.sync_copy(rows_v1, acc.at[dst_v.at[j0 + 1]], add=True)
            return _

        lax.fori_loop(0, SEG // 2, edge_body, None)
        return _

    lax.fori_loop(0, NSEG, seg_body, None)

    plsc.subcore_barrier()

    out_base = cid * R + base
    pltpu.sync_copy(acc.at[pl.ds(base, ROWS_PER_TILE)],
                    agg_hbm.at[pl.ds(out_base, ROWS_PER_TILE)])


_sc_propagate = pl.kernel(
    _sc_propagate_body,
    out_type=jax.ShapeDtypeStruct((NC * R, HH), jnp.float32),
    mesh=_MESH,
    scratch_types=[
        pltpu.VMEM((SEG, CHUNK), jnp.int32),
        pltpu.VMEM((SEG, CHUNK), jnp.int32),
        pltpu.VMEM((CHUNK, HH), jnp.float32),
        pltpu.VMEM((CHUNK, HH), jnp.float32),
        pltpu.VMEM_SHARED((R, HH), jnp.float32),
        pltpu.SemaphoreType.DMA,
        pltpu.SemaphoreType.DMA,
    ],
    compiler_params=_SC_PARAMS,
)


_BM = 1024
_GRID = R // _BM


def _dot(a, b):
    return jnp.dot(a, b, preferred_element_type=jnp.float32,
                   precision=lax.Precision.DEFAULT)


def _tc_finalize_deg_body(dego_ref, degi_ref, dinv_out_ref, dinv_in_ref):
    do = jnp.maximum(jnp.sum(dego_ref[...], axis=0), 1.0)
    di = jnp.maximum(jnp.sum(degi_ref[...], axis=0), 1.0)
    dinv_out_ref[...] = lax.rsqrt(do)[:, None]
    dinv_in_ref[...] = lax.rsqrt(di)[:, None]


def _tc_finalize_deg(dego_p, degi_p):
    return pl.pallas_call(
        _tc_finalize_deg_body,
        out_shape=(
            jax.ShapeDtypeStruct((R, 1), jnp.float32),
            jax.ShapeDtypeStruct((R, 1), jnp.float32),
        ),
    )(dego_p, degi_p)


def _tc_y1_body(x_ref, w_ref, dinv_ref, out_ref):
    y = _dot(x_ref[...], w_ref[...]) * dinv_ref[...]
    out_ref[0, :, :] = y[:, :HH]
    out_ref[1, :, :] = y[:, HH:]


def _tc_y1(feats_p, W_gc1, dinv_out):
    return pl.pallas_call(
        _tc_y1_body,
        grid=(_GRID,),
        in_specs=[
            pl.BlockSpec((_BM, D), lambda r: (r, 0)),
            pl.BlockSpec((D, H), lambda r: (0, 0)),
            pl.BlockSpec((_BM, 1), lambda r: (r, 0)),
        ],
        out_specs=pl.BlockSpec((NC, _BM, HH), lambda r: (0, r, 0)),
        out_shape=jax.ShapeDtypeStruct((NC, R, HH), jnp.float32),
    )(feats_p, W_gc1, dinv_out)


def _tc_mid_body(agg_ref, din_ref, dout_ref, b_ref, out_ref):
    x = jax.nn.relu(agg_ref[...] * din_ref[...][None] + b_ref[...])
    out_ref[...] = x * dout_ref[...][None]


def _tc_mid(agg1, dinv_in, dinv_out, b_gc1_2):
    return pl.pallas_call(
        _tc_mid_body,
        grid=(_GRID,),
        in_specs=[
            pl.BlockSpec((NC, _BM, HH), lambda r: (0, r, 0)),
            pl.BlockSpec((_BM, 1), lambda r: (r, 0)),
            pl.BlockSpec((_BM, 1), lambda r: (r, 0)),
            pl.BlockSpec((NC, 1, HH), lambda r: (0, 0, 0)),
        ],
        out_specs=pl.BlockSpec((NC, _BM, HH), lambda r: (0, r, 0)),
        out_shape=jax.ShapeDtypeStruct((NC, R, HH), jnp.float32),
    )(agg1, dinv_in, dinv_out, b_gc1_2)


def _tc_final_body(agg_ref, din_ref, feat_ref, wg2_ref, bg2_ref,
                   wm1_ref, bm1_ref, wm2_ref, bm2_ref, out_ref):
    a = jnp.concatenate([agg_ref[0], agg_ref[1]], axis=1) * din_ref[...]
    gcn = _dot(a, wg2_ref[...]) + bg2_ref[...] + feat_ref[...]
    m = jax.nn.relu(_dot(gcn, wm1_ref[...]) + bm1_ref[...])
    out_ref[...] = _dot(m, wm2_ref[...]) + bm2_ref[...] + gcn


def _tc_final(agg2, dinv_in, feats_p, W_gc2, b_gc2, W_m1, b_m1, W_m2, b_m2):
    return pl.pallas_call(
        _tc_final_body,
        grid=(_GRID,),
        in_specs=[
            pl.BlockSpec((NC, _BM, HH), lambda r: (0, r, 0)),
            pl.BlockSpec((_BM, 1), lambda r: (r, 0)),
            pl.BlockSpec((_BM, D), lambda r: (r, 0)),
            pl.BlockSpec((H, D), lambda r: (0, 0)),
            pl.BlockSpec((1, D), lambda r: (0, 0)),
            pl.BlockSpec((D, H), lambda r: (0, 0)),
            pl.BlockSpec((1, H), lambda r: (0, 0)),
            pl.BlockSpec((H, D), lambda r: (0, 0)),
            pl.BlockSpec((1, D), lambda r: (0, 0)),
        ],
        out_specs=pl.BlockSpec((_BM, D), lambda r: (r, 0)),
        out_shape=jax.ShapeDtypeStruct((R, D), jnp.float32),
    )(agg2, dinv_in, feats_p, W_gc2, b_gc2, W_m1, b_m1, W_m2, b_m2)


def kernel(features, edge_index, W_gc1, b_gc1, W_gc2, b_gc2,
           W_m1, b_m1, W_m2, b_m2):
    src = edge_index[0]
    dst = edge_index[1]
    pad = jnp.full((E_PAD - E,), N, dtype=jnp.int32)
    src_p = jnp.concatenate([src, pad])
    dst_p = jnp.concatenate([dst, pad])

    srcd = src_p.reshape(NC * NS, EPT_DEG)
    dstd = dst_p.reshape(NC * NS, EPT_DEG)
    srcp0 = src_p.reshape(NS, NSEG, SEG, CHUNK)
    srcp1 = srcp0 + R
    dstp = dst_p.reshape(NS, NSEG, SEG, CHUNK)

    feats_p = jnp.pad(features, ((0, R - N), (0, 0)))

    dego_p, degi_p = _sc_degrees(srcd, dstd)
    dinv_out, dinv_in = _tc_finalize_deg(dego_p, degi_p)

    y1 = _tc_y1(feats_p, W_gc1, dinv_out)
    agg1 = _sc_propagate(y1.reshape(NC * R, HH), srcp0, srcp1, dstp)
    h2 = _tc_mid(agg1.reshape(NC, R, HH), dinv_in, dinv_out,
                 b_gc1.reshape(NC, 1, HH))
    agg2 = _sc_propagate(h2.reshape(NC * R, HH), srcp0, srcp1, dstp)
    out_p = _tc_final(agg2.reshape(NC, R, HH), dinv_in, feats_p,
                      W_gc2, b_gc2.reshape(1, D), W_m1, b_m1.reshape(1, H),
                      W_m2, b_m2.reshape(1, D))
    return out_p[:N]
